# speedup vs baseline: 1.4331x; 1.4331x over previous
import functools

import jax
import jax.numpy as jnp
from jax import lax
from jax.experimental import pallas as pl
from jax.experimental.pallas import tpu as pltpu
from jax.experimental.pallas import tpu_sc as plsc

VOCAB = 100000
HIDDEN = 1024
B = 4
S = 4096
TOT = B * S
NUM_WORKERS = 32
ROWS_PER_W = TOT // NUM_WORKERS
CHUNK = 32
NCHUNK = ROWS_PER_W // CHUNK


def _embed_call(flat_ids, table):
    mesh = plsc.VectorSubcoreMesh(core_axis_name="c", subcore_axis_name="s")

    @functools.partial(
        pl.kernel,
        mesh=mesh,
        out_type=jax.ShapeDtypeStruct((TOT, HIDDEN), jnp.float32),
        scratch_types=[
            pltpu.VMEM((ROWS_PER_W,), jnp.int32),
            pltpu.VMEM((CHUNK, HIDDEN), jnp.float32),
            pltpu.SemaphoreType.DMA,
        ],
    )
    def gather_kernel(ids_hbm, table_hbm, out_hbm, idx_v, rows_v, gsem):
        wid = lax.axis_index("s") * 2 + lax.axis_index("c")
        base = wid * ROWS_PER_W
        pltpu.sync_copy(ids_hbm.at[pl.ds(base, ROWS_PER_W)], idx_v)

        def body(j, carry):
            off = j * CHUNK
            pltpu.async_copy(
                table_hbm.at[idx_v.at[pl.ds(off, CHUNK)]], rows_v, gsem
            ).wait()
            pltpu.sync_copy(rows_v, out_hbm.at[pl.ds(base + off, CHUNK)])
            return carry

        lax.fori_loop(0, NCHUNK, body, 0)

    return gather_kernel(flat_ids, table)


def kernel(input_ids, attention_mask, table):
    flat_ids = input_ids.reshape(TOT)
    out = _embed_call(flat_ids, table)
    return (out.reshape(B, S, HIDDEN), attention_mask)


# speedup vs baseline: 1.6393x; 1.1439x over previous
import functools

import jax
import jax.numpy as jnp
from jax import lax
from jax.experimental import pallas as pl
from jax.experimental.pallas import tpu as pltpu
from jax.experimental.pallas import tpu_sc as plsc

VOCAB = 100000
HIDDEN = 1024
B = 4
S = 4096
TOT = B * S
NUM_WORKERS = 32
ROWS_PER_W = TOT // NUM_WORKERS
CHUNK = 32
NCHUNK = ROWS_PER_W // CHUNK
NBUF = 2


def _embed_call(flat_ids, table):
    mesh = plsc.VectorSubcoreMesh(core_axis_name="c", subcore_axis_name="s")

    @functools.partial(
        pl.kernel,
        mesh=mesh,
        out_type=jax.ShapeDtypeStruct((TOT, HIDDEN), jnp.float32),
        scratch_types=[
            pltpu.VMEM((ROWS_PER_W,), jnp.int32),
            pltpu.VMEM((NBUF, CHUNK, HIDDEN), jnp.float32),
            pltpu.SemaphoreType.DMA,
            pltpu.SemaphoreType.DMA,
            pltpu.SemaphoreType.DMA,
            pltpu.SemaphoreType.DMA,
        ],
    )
    def gather_kernel(ids_hbm, table_hbm, out_hbm, idx_v, rows_v, g0, g1, s0, s1):
        gsem = (g0, g1)
        osem = (s0, s1)
        wid = lax.axis_index("s") * 2 + lax.axis_index("c")
        base = wid * ROWS_PER_W
        pltpu.sync_copy(ids_hbm.at[pl.ds(base, ROWS_PER_W)], idx_v)

        def gather_args(j, b):
            return (table_hbm.at[idx_v.at[pl.ds(j * CHUNK, CHUNK)]],
                    rows_v.at[b], gsem[b])

        def store_args(j, b):
            return (rows_v.at[b],
                    out_hbm.at[pl.ds(base + j * CHUNK, CHUNK)], osem[b])

        for b in range(NBUF):
            pltpu.async_copy(*gather_args(b, b))

        def body(i, carry):
            for b in range(NBUF):
                j = i * NBUF + b
                pltpu.make_async_copy(*gather_args(j, b)).wait()
                pltpu.async_copy(*store_args(j, b))
                pltpu.make_async_copy(*store_args(j, b)).wait()
                pltpu.async_copy(*gather_args(j + NBUF, b))
            return carry

        lax.fori_loop(0, NCHUNK // NBUF - 1, body, 0)

        for b in range(NBUF):
            j = NCHUNK - NBUF + b
            pltpu.make_async_copy(*gather_args(j, b)).wait()
            pltpu.async_copy(*store_args(j, b))
        for b in range(NBUF):
            j = NCHUNK - NBUF + b
            pltpu.make_async_copy(*store_args(j, b)).wait()

    return gather_kernel(flat_ids, table)


def kernel(input_ids, attention_mask, table):
    flat_ids = input_ids.reshape(TOT)
    out = _embed_call(flat_ids, table)
    return (out.reshape(B, S, HIDDEN), attention_mask)


# speedup vs baseline: 1.6590x; 1.0120x over previous
import functools

import jax
import jax.numpy as jnp
from jax import lax
from jax.experimental import pallas as pl
from jax.experimental.pallas import tpu as pltpu
from jax.experimental.pallas import tpu_sc as plsc

VOCAB = 100000
HIDDEN = 1024
B = 4
S = 4096
TOT = B * S
NUM_WORKERS = 32
ROWS_PER_W = TOT // NUM_WORKERS
CHUNK = 16
NCHUNK = ROWS_PER_W // CHUNK
NBUF = 4


def _embed_call(flat_ids, table):
    mesh = plsc.VectorSubcoreMesh(core_axis_name="c", subcore_axis_name="s")

    @functools.partial(
        pl.kernel,
        mesh=mesh,
        out_type=jax.ShapeDtypeStruct((TOT, HIDDEN), jnp.float32),
        scratch_types=[
            pltpu.VMEM((ROWS_PER_W,), jnp.int32),
            pltpu.VMEM((NBUF, CHUNK, HIDDEN), jnp.float32),
        ] + [pltpu.SemaphoreType.DMA] * (2 * NBUF),
    )
    def gather_kernel(ids_hbm, table_hbm, out_hbm, idx_v, rows_v, *sems):
        gsem = sems[:NBUF]
        osem = sems[NBUF:]
        wid = lax.axis_index("s") * 2 + lax.axis_index("c")
        base = wid * ROWS_PER_W
        pltpu.sync_copy(ids_hbm.at[pl.ds(base, ROWS_PER_W)], idx_v)

        def gather_args(j, b):
            return (table_hbm.at[idx_v.at[pl.ds(j * CHUNK, CHUNK)]],
                    rows_v.at[b], gsem[b])

        def store_args(j, b):
            return (rows_v.at[b],
                    out_hbm.at[pl.ds(base + j * CHUNK, CHUNK)], osem[b])

        for b in range(NBUF):
            pltpu.async_copy(*gather_args(b, b))

        def body(i, carry):
            for b in range(NBUF):
                j = i * NBUF + b
                pltpu.make_async_copy(*gather_args(j, b)).wait()
                pltpu.async_copy(*store_args(j, b))
                pltpu.make_async_copy(*store_args(j, b)).wait()
                pltpu.async_copy(*gather_args(j + NBUF, b))
            return carry

        lax.fori_loop(0, NCHUNK // NBUF - 1, body, 0)

        for b in range(NBUF):
            j = NCHUNK - NBUF + b
            pltpu.make_async_copy(*gather_args(j, b)).wait()
            pltpu.async_copy(*store_args(j, b))
        for b in range(NBUF):
            j = NCHUNK - NBUF + b
            pltpu.make_async_copy(*store_args(j, b)).wait()

    return gather_kernel(flat_ids, table)


def kernel(input_ids, attention_mask, table):
    flat_ids = input_ids.reshape(TOT)
    out = _embed_call(flat_ids, table)
    return (out.reshape(B, S, HIDDEN), attention_mask)
